# Initial kernel scaffold; baseline (speedup 1.0000x reference)
import jax
import jax.numpy as jnp
from jax.experimental import pallas as pl


def kernel(x, index, dim, dim_size, W, learnable_param):
    raise NotImplementedError("write your pallas kernel here")


# speedup vs baseline: 4.9772x; 4.9772x over previous
import functools

import jax
import jax.numpy as jnp
from jax import lax
from jax.experimental import pallas as pl
from jax.experimental.pallas import tpu as pltpu
from jax.experimental.pallas import tpu_sc as plsc

E = 320000
D = 128
PD = 64
N = 10000

NC = 2
NS = 16
NW = NC * NS
EPW = E // NW
CHUNK = 80
NCHUNK = EPW // CHUNK
RPS = 624
TAIL = N - NS * RPS

GATE_B = 2000


def _gate_body(lp_ref, x_ref, wt_ref, out_ref):
    x = x_ref[...]
    a = x[:, :PD]
    b = x[:, PD:]
    wt = wt_ref[...]
    ga = jnp.clip(lax.dot(a, wt, preferred_element_type=jnp.float32), 0.0, 1.0)
    gb = jnp.clip(lax.dot(b, wt, preferred_element_type=jnp.float32), 0.0, 1.0)
    at = jnp.concatenate([a, a], axis=1)
    bt = jnp.concatenate([b, b], axis=1)
    lp = jnp.abs(lp_ref[0])
    out_ref[...] = lp * (at * ga + bt * gb) + x


def _gate(x, wt, lp):
    return pl.pallas_call(
        _gate_body,
        grid=(E // GATE_B,),
        in_specs=[
            pl.BlockSpec(memory_space=pltpu.SMEM),
            pl.BlockSpec((GATE_B, D), lambda i: (i, 0)),
            pl.BlockSpec((PD, D), lambda i: (0, 0)),
        ],
        out_specs=pl.BlockSpec((GATE_B, D), lambda i: (i, 0)),
        out_shape=jax.ShapeDtypeStruct((E, D), jnp.float32),
    )(lp, x, wt)


def _sc_scatter(comb, idx, zeros):
    mesh = plsc.VectorSubcoreMesh(core_axis_name="c", subcore_axis_name="s")

    @functools.partial(
        pl.kernel,
        mesh=mesh,
        out_type=jax.ShapeDtypeStruct((NC * N, D), jnp.float32),
        scratch_types=[
            pltpu.VMEM((CHUNK, D), jnp.float32),
            pltpu.VMEM((CHUNK,), jnp.int32),
            pltpu.VMEM_SHARED((N, D), jnp.float32),
        ],
    )
    def run(comb_hbm, idx_hbm, zeros_hbm, out_hbm, ebuf, ibuf, acc):
        c = lax.axis_index("c")
        s = lax.axis_index("s")
        wid = s * NC + c
        pltpu.sync_copy(zeros_hbm.at[pl.ds(s * RPS, RPS)],
                        acc.at[pl.ds(s * RPS, RPS)])

        @pl.when(s == 0)
        def _():
            pltpu.sync_copy(zeros_hbm.at[pl.ds(NS * RPS, TAIL)],
                            acc.at[pl.ds(NS * RPS, TAIL)])

        plsc.subcore_barrier()
        base = wid * EPW

        def body(i, carry):
            off = base + i * CHUNK
            pltpu.sync_copy(comb_hbm.at[pl.ds(off, CHUNK)], ebuf)
            pltpu.sync_copy(idx_hbm.at[pl.ds(off, CHUNK)], ibuf)
            pltpu.sync_copy(ebuf, acc.at[ibuf], add=True)
            return carry

        lax.fori_loop(0, NCHUNK, body, 0)
        plsc.subcore_barrier()
        pltpu.sync_copy(acc.at[pl.ds(s * RPS, RPS)],
                        out_hbm.at[pl.ds(c * N + s * RPS, RPS)])

        @pl.when(s == 0)
        def _():
            pltpu.sync_copy(acc.at[pl.ds(NS * RPS, TAIL)],
                            out_hbm.at[pl.ds(c * N + NS * RPS, TAIL)])

    return run(comb, idx, zeros)


def _add_body(p_ref, q_ref, o_ref):
    o_ref[...] = p_ref[...] + q_ref[...]


def _final_add(partials):
    bn = 2000
    return pl.pallas_call(
        _add_body,
        grid=(N // bn,),
        in_specs=[
            pl.BlockSpec((bn, D), lambda i: (i, 0)),
            pl.BlockSpec((bn, D), lambda i: (i + N // bn, 0)),
        ],
        out_specs=pl.BlockSpec((bn, D), lambda i: (i, 0)),
        out_shape=jax.ShapeDtypeStruct((N, D), jnp.float32),
    )(partials, partials)


def kernel(x, index, dim, dim_size, W, learnable_param):
    del dim, dim_size
    wt = W.T
    comb = _gate(x, wt, learnable_param)
    idx = index.astype(jnp.int32)
    zeros = jnp.zeros((N, D), jnp.float32)
    partials = _sc_scatter(comb, idx, zeros)
    return _final_add(partials)


# speedup vs baseline: 7.3926x; 1.4853x over previous
import functools

import jax
import jax.numpy as jnp
from jax import lax
from jax.experimental import pallas as pl
from jax.experimental.pallas import tpu as pltpu
from jax.experimental.pallas import tpu_sc as plsc

E = 320000
D = 128
PD = 64
N = 10000

NC = 2
NS = 16
NW = NC * NS
EPW = E // NW
CHUNK = 40
NCHUNK = EPW // CHUNK
RPS = 624
TAIL = N - NS * RPS

GATE_B = 2000


def _gate_body(lp_ref, x_ref, wt_ref, out_ref):
    x = x_ref[...]
    a = x[:, :PD]
    b = x[:, PD:]
    wt = wt_ref[...]
    ga = jnp.clip(lax.dot(a, wt, preferred_element_type=jnp.float32), 0.0, 1.0)
    gb = jnp.clip(lax.dot(b, wt, preferred_element_type=jnp.float32), 0.0, 1.0)
    at = jnp.concatenate([a, a], axis=1)
    bt = jnp.concatenate([b, b], axis=1)
    lp = jnp.abs(lp_ref[0])
    out_ref[...] = lp * (at * ga + bt * gb) + x


def _gate(x, wt, lp):
    return pl.pallas_call(
        _gate_body,
        grid=(E // GATE_B,),
        in_specs=[
            pl.BlockSpec(memory_space=pltpu.SMEM),
            pl.BlockSpec((GATE_B, D), lambda i: (i, 0)),
            pl.BlockSpec((PD, D), lambda i: (0, 0)),
        ],
        out_specs=pl.BlockSpec((GATE_B, D), lambda i: (i, 0)),
        out_shape=jax.ShapeDtypeStruct((E, D), jnp.float32),
    )(lp, x, wt)


NBUF = 5


def _sc_scatter(comb, idx, zeros):
    mesh = plsc.VectorSubcoreMesh(core_axis_name="c", subcore_axis_name="s")

    @functools.partial(
        pl.kernel,
        mesh=mesh,
        out_type=jax.ShapeDtypeStruct((NC * N, D), jnp.float32),
        scratch_types=(
            [pltpu.VMEM((CHUNK, D), jnp.float32) for _ in range(NBUF)]
            + [pltpu.VMEM((CHUNK,), jnp.int32) for _ in range(NBUF)]
            + [pltpu.VMEM_SHARED((N, D), jnp.float32)]
            + [pltpu.SemaphoreType.DMA for _ in range(NBUF)]
        ),
    )
    def run(comb_hbm, idx_hbm, zeros_hbm, out_hbm,
            eb0, eb1, eb2, eb3, eb4, ib0, ib1, ib2, ib3, ib4,
            acc, sg0, sg1, sg2, sg3, sg4):
        ebufs = [eb0, eb1, eb2, eb3, eb4]
        ibufs = [ib0, ib1, ib2, ib3, ib4]
        sgs = [sg0, sg1, sg2, sg3, sg4]
        c = lax.axis_index("c")
        s = lax.axis_index("s")
        wid = s * NC + c
        base = wid * EPW

        for b in range(NBUF):
            off = base + b * CHUNK
            pltpu.async_copy(comb_hbm.at[pl.ds(off, CHUNK)], ebufs[b], sgs[b])
            pltpu.async_copy(idx_hbm.at[pl.ds(off, CHUNK)], ibufs[b], sgs[b])

        pltpu.sync_copy(zeros_hbm.at[pl.ds(s * RPS, RPS)],
                        acc.at[pl.ds(s * RPS, RPS)])

        @pl.when(s == 0)
        def _():
            pltpu.sync_copy(zeros_hbm.at[pl.ds(NS * RPS, TAIL)],
                            acc.at[pl.ds(NS * RPS, TAIL)])

        plsc.subcore_barrier()

        def body(g, carry):
            for b in range(NBUF):
                i = g * NBUF + b
                pltpu.make_async_copy(
                    comb_hbm.at[pl.ds(base, CHUNK)], ebufs[b], sgs[b]).wait()
                pltpu.make_async_copy(
                    idx_hbm.at[pl.ds(base, CHUNK)], ibufs[b], sgs[b]).wait()
                pltpu.sync_copy(ebufs[b], acc.at[ibufs[b]], add=True)
                nxt = i + NBUF

                @pl.when(nxt < NCHUNK)
                def _():
                    off = base + nxt * CHUNK
                    pltpu.async_copy(
                        comb_hbm.at[pl.ds(off, CHUNK)], ebufs[b], sgs[b])
                    pltpu.async_copy(
                        idx_hbm.at[pl.ds(off, CHUNK)], ibufs[b], sgs[b])
            return carry

        lax.fori_loop(0, NCHUNK // NBUF, body, 0)
        plsc.subcore_barrier()
        pltpu.sync_copy(acc.at[pl.ds(s * RPS, RPS)],
                        out_hbm.at[pl.ds(c * N + s * RPS, RPS)])

        @pl.when(s == 0)
        def _():
            pltpu.sync_copy(acc.at[pl.ds(NS * RPS, TAIL)],
                            out_hbm.at[pl.ds(c * N + NS * RPS, TAIL)])

    return run(comb, idx, zeros)


def _add_body(p_ref, q_ref, o_ref):
    o_ref[...] = p_ref[...] + q_ref[...]


def _final_add(partials):
    bn = 2000
    return pl.pallas_call(
        _add_body,
        grid=(N // bn,),
        in_specs=[
            pl.BlockSpec((bn, D), lambda i: (i, 0)),
            pl.BlockSpec((bn, D), lambda i: (i + N // bn, 0)),
        ],
        out_specs=pl.BlockSpec((bn, D), lambda i: (i, 0)),
        out_shape=jax.ShapeDtypeStruct((N, D), jnp.float32),
    )(partials, partials)


def kernel(x, index, dim, dim_size, W, learnable_param):
    del dim, dim_size
    wt = W.T
    comb = _gate(x, wt, learnable_param)
    idx = index.astype(jnp.int32)
    zeros = jnp.zeros((N, D), jnp.float32)
    partials = _sc_scatter(comb, idx, zeros)
    return _final_add(partials)


# speedup vs baseline: 8.1001x; 1.0957x over previous
import functools

import jax
import jax.numpy as jnp
from jax import lax
from jax.experimental import pallas as pl
from jax.experimental.pallas import tpu as pltpu
from jax.experimental.pallas import tpu_sc as plsc

E = 320000
D = 128
PD = 64
N = 10000

NSEG = 2
ESEG = E // NSEG
NC = 2
NS = 16
NW = NC * NS
EPW = ESEG // NW
CHUNK = 40
NCHUNK = EPW // CHUNK
NBUF = 5
RPS = 624
TAIL = N - NS * RPS

GATE_B = 2000


def _gate_body(lp_ref, x_ref, wt_ref, out_ref):
    x = x_ref[...]
    a = x[:, :PD]
    b = x[:, PD:]
    wt = wt_ref[...]
    ga = jnp.clip(lax.dot(a, wt, preferred_element_type=jnp.float32), 0.0, 1.0)
    gb = jnp.clip(lax.dot(b, wt, preferred_element_type=jnp.float32), 0.0, 1.0)
    at = jnp.concatenate([a, a], axis=1)
    bt = jnp.concatenate([b, b], axis=1)
    lp = jnp.abs(lp_ref[0])
    out_ref[...] = lp * (at * ga + bt * gb) + x


def _gate(x, wt, lp, seg):
    nblk = ESEG // GATE_B
    return pl.pallas_call(
        _gate_body,
        grid=(nblk,),
        in_specs=[
            pl.BlockSpec(memory_space=pltpu.SMEM),
            pl.BlockSpec((GATE_B, D), lambda i, _o=seg * nblk: (i + _o, 0)),
            pl.BlockSpec((PD, D), lambda i: (0, 0)),
        ],
        out_specs=pl.BlockSpec((GATE_B, D), lambda i: (i, 0)),
        out_shape=jax.ShapeDtypeStruct((ESEG, D), jnp.float32),
    )(lp, x, wt)


def _sc_scatter(comb, idx, zeros, seg):
    mesh = plsc.VectorSubcoreMesh(core_axis_name="c", subcore_axis_name="s")
    ibase0 = seg * ESEG

    @functools.partial(
        pl.kernel,
        mesh=mesh,
        out_type=jax.ShapeDtypeStruct((NC * N, D), jnp.float32),
        scratch_types=(
            [pltpu.VMEM((CHUNK, D), jnp.float32) for _ in range(NBUF)]
            + [pltpu.VMEM((CHUNK,), jnp.int32) for _ in range(NBUF)]
            + [pltpu.VMEM_SHARED((N, D), jnp.float32)]
            + [pltpu.SemaphoreType.DMA for _ in range(NBUF)]
        ),
    )
    def run(comb_hbm, idx_hbm, zeros_hbm, out_hbm,
            eb0, eb1, eb2, eb3, eb4, ib0, ib1, ib2, ib3, ib4,
            acc, sg0, sg1, sg2, sg3, sg4):
        ebufs = [eb0, eb1, eb2, eb3, eb4]
        ibufs = [ib0, ib1, ib2, ib3, ib4]
        sgs = [sg0, sg1, sg2, sg3, sg4]
        c = lax.axis_index("c")
        s = lax.axis_index("s")
        wid = s * NC + c
        base = wid * EPW
        ibase = ibase0 + base

        for b in range(NBUF):
            off = b * CHUNK
            pltpu.async_copy(comb_hbm.at[pl.ds(base + off, CHUNK)],
                             ebufs[b], sgs[b])
            pltpu.async_copy(idx_hbm.at[pl.ds(ibase + off, CHUNK)],
                             ibufs[b], sgs[b])

        pltpu.sync_copy(zeros_hbm.at[pl.ds(s * RPS, RPS)],
                        acc.at[pl.ds(s * RPS, RPS)])

        @pl.when(s == 0)
        def _():
            pltpu.sync_copy(zeros_hbm.at[pl.ds(NS * RPS, TAIL)],
                            acc.at[pl.ds(NS * RPS, TAIL)])

        plsc.subcore_barrier()

        def body(g, carry):
            for b in range(NBUF):
                i = g * NBUF + b
                pltpu.make_async_copy(
                    comb_hbm.at[pl.ds(base, CHUNK)], ebufs[b], sgs[b]).wait()
                pltpu.make_async_copy(
                    idx_hbm.at[pl.ds(ibase, CHUNK)], ibufs[b], sgs[b]).wait()
                pltpu.sync_copy(ebufs[b], acc.at[ibufs[b]], add=True)
                nxt = i + NBUF

                @pl.when(nxt < NCHUNK)
                def _():
                    off = nxt * CHUNK
                    pltpu.async_copy(comb_hbm.at[pl.ds(base + off, CHUNK)],
                                     ebufs[b], sgs[b])
                    pltpu.async_copy(idx_hbm.at[pl.ds(ibase + off, CHUNK)],
                                     ibufs[b], sgs[b])
            return carry

        lax.fori_loop(0, NCHUNK // NBUF, body, 0)
        plsc.subcore_barrier()
        pltpu.sync_copy(acc.at[pl.ds(s * RPS, RPS)],
                        out_hbm.at[pl.ds(c * N + s * RPS, RPS)])

        @pl.when(s == 0)
        def _():
            pltpu.sync_copy(acc.at[pl.ds(NS * RPS, TAIL)],
                            out_hbm.at[pl.ds(c * N + NS * RPS, TAIL)])

    return run(comb, idx, zeros)


def _add_body(*refs):
    o_ref = refs[-1]
    acc = refs[0][...]
    for r in refs[1:-1]:
        acc = acc + r[...]
    o_ref[...] = acc


def _final_add(partials_list):
    bn = 2000
    nblk = N // bn
    in_specs = []
    args = []
    for p in partials_list:
        in_specs.append(pl.BlockSpec((bn, D), lambda i: (i, 0)))
        in_specs.append(pl.BlockSpec((bn, D), lambda i, _o=nblk: (i + _o, 0)))
        args += [p, p]
    return pl.pallas_call(
        _add_body,
        grid=(nblk,),
        in_specs=in_specs,
        out_specs=pl.BlockSpec((bn, D), lambda i: (i, 0)),
        out_shape=jax.ShapeDtypeStruct((N, D), jnp.float32),
    )(*args)


def kernel(x, index, dim, dim_size, W, learnable_param):
    del dim, dim_size
    wt = W.T
    idx = index.astype(jnp.int32)
    zeros = jnp.zeros((N, D), jnp.float32)
    partials = []
    for k in range(NSEG):
        comb_k = _gate(x, wt, learnable_param, k)
        partials.append(_sc_scatter(comb_k, idx, zeros, k))
    return _final_add(partials)


# speedup vs baseline: 9.7813x; 1.2076x over previous
import functools

import jax
import jax.numpy as jnp
from jax import lax
from jax.experimental import pallas as pl
from jax.experimental.pallas import tpu as pltpu
from jax.experimental.pallas import tpu_sc as plsc

E = 320000
D = 128
PD = 64
N = 10000

NSEG = 2
ESEG = E // NSEG
NC = 2
NS = 16
NW = NC * NS
EPW = ESEG // NW
CHUNK = 40
NCHUNK = EPW // CHUNK
NBUF = 5
RPS = 624
TAIL = N - NS * RPS

GATE_B = 8000


def _gate_body(lp_ref, x_ref, wt_ref, out_ref):
    x = x_ref[...]
    a = x[:, :PD]
    b = x[:, PD:]
    wt = wt_ref[...]
    ga = jnp.clip(lax.dot(a, wt, preferred_element_type=jnp.float32), 0.0, 1.0)
    gb = jnp.clip(lax.dot(b, wt, preferred_element_type=jnp.float32), 0.0, 1.0)
    at = jnp.concatenate([a, a], axis=1)
    bt = jnp.concatenate([b, b], axis=1)
    lp = jnp.abs(lp_ref[0])
    out_ref[...] = lp * (at * ga + bt * gb) + x


def _gate(x, wt, lp, seg):
    nblk = ESEG // GATE_B
    return pl.pallas_call(
        _gate_body,
        grid=(nblk,),
        in_specs=[
            pl.BlockSpec(memory_space=pltpu.SMEM),
            pl.BlockSpec((GATE_B, D), lambda i, _o=seg * nblk: (i + _o, 0)),
            pl.BlockSpec((PD, D), lambda i: (0, 0)),
        ],
        out_specs=pl.BlockSpec((GATE_B, D), lambda i: (i, 0)),
        out_shape=jax.ShapeDtypeStruct((ESEG, D), jnp.float32),
    )(lp, x, wt)


def _sc_scatter(comb, idx, zeros, seg):
    mesh = plsc.VectorSubcoreMesh(core_axis_name="c", subcore_axis_name="s")
    ibase0 = seg * ESEG

    @functools.partial(
        pl.kernel,
        mesh=mesh,
        out_type=jax.ShapeDtypeStruct((NC * N, D), jnp.float32),
        scratch_types=(
            [pltpu.VMEM((CHUNK, D), jnp.float32) for _ in range(NBUF)]
            + [pltpu.VMEM((CHUNK,), jnp.int32) for _ in range(NBUF)]
            + [pltpu.VMEM_SHARED((N, D), jnp.float32)]
            + [pltpu.SemaphoreType.DMA for _ in range(NBUF)]
        ),
    )
    def run(comb_hbm, idx_hbm, zeros_hbm, out_hbm,
            eb0, eb1, eb2, eb3, eb4, ib0, ib1, ib2, ib3, ib4,
            acc, sg0, sg1, sg2, sg3, sg4):
        ebufs = [eb0, eb1, eb2, eb3, eb4]
        ibufs = [ib0, ib1, ib2, ib3, ib4]
        sgs = [sg0, sg1, sg2, sg3, sg4]
        c = lax.axis_index("c")
        s = lax.axis_index("s")
        wid = s * NC + c
        base = wid * EPW
        ibase = ibase0 + base

        for b in range(NBUF):
            off = b * CHUNK
            pltpu.async_copy(comb_hbm.at[pl.ds(base + off, CHUNK)],
                             ebufs[b], sgs[b])
            pltpu.async_copy(idx_hbm.at[pl.ds(ibase + off, CHUNK)],
                             ibufs[b], sgs[b])

        pltpu.sync_copy(zeros_hbm.at[pl.ds(s * RPS, RPS)],
                        acc.at[pl.ds(s * RPS, RPS)])

        @pl.when(s == 0)
        def _():
            pltpu.sync_copy(zeros_hbm.at[pl.ds(NS * RPS, TAIL)],
                            acc.at[pl.ds(NS * RPS, TAIL)])

        plsc.subcore_barrier()

        def body(g, carry):
            for b in range(NBUF):
                i = g * NBUF + b
                pltpu.make_async_copy(
                    comb_hbm.at[pl.ds(base, CHUNK)], ebufs[b], sgs[b]).wait()
                pltpu.make_async_copy(
                    idx_hbm.at[pl.ds(ibase, CHUNK)], ibufs[b], sgs[b]).wait()
                pltpu.sync_copy(ebufs[b], acc.at[ibufs[b]], add=True)
                nxt = i + NBUF

                @pl.when(nxt < NCHUNK)
                def _():
                    off = nxt * CHUNK
                    pltpu.async_copy(comb_hbm.at[pl.ds(base + off, CHUNK)],
                                     ebufs[b], sgs[b])
                    pltpu.async_copy(idx_hbm.at[pl.ds(ibase + off, CHUNK)],
                                     ibufs[b], sgs[b])
            return carry

        lax.fori_loop(0, NCHUNK // NBUF, body, 0)
        plsc.subcore_barrier()
        pltpu.sync_copy(acc.at[pl.ds(s * RPS, RPS)],
                        out_hbm.at[pl.ds(c * N + s * RPS, RPS)])

        @pl.when(s == 0)
        def _():
            pltpu.sync_copy(acc.at[pl.ds(NS * RPS, TAIL)],
                            out_hbm.at[pl.ds(c * N + NS * RPS, TAIL)])

    return run(comb, idx, zeros)


def _add_body(*refs):
    o_ref = refs[-1]
    acc = refs[0][...]
    for r in refs[1:-1]:
        acc = acc + r[...]
    o_ref[...] = acc


def _final_add(partials_list):
    bn = 2000
    nblk = N // bn
    in_specs = []
    args = []
    for p in partials_list:
        in_specs.append(pl.BlockSpec((bn, D), lambda i: (i, 0)))
        in_specs.append(pl.BlockSpec((bn, D), lambda i, _o=nblk: (i + _o, 0)))
        args += [p, p]
    return pl.pallas_call(
        _add_body,
        grid=(nblk,),
        in_specs=in_specs,
        out_specs=pl.BlockSpec((bn, D), lambda i: (i, 0)),
        out_shape=jax.ShapeDtypeStruct((N, D), jnp.float32),
    )(*args)


def kernel(x, index, dim, dim_size, W, learnable_param):
    del dim, dim_size
    wt = W.T
    idx = index.astype(jnp.int32)
    zeros = jnp.zeros((N, D), jnp.float32)
    partials = []
    for k in range(NSEG):
        comb_k = _gate(x, wt, learnable_param, k)
        partials.append(_sc_scatter(comb_k, idx, zeros, k))
    return _final_add(partials)


# speedup vs baseline: 9.8749x; 1.0096x over previous
import functools

import jax
import jax.numpy as jnp
from jax import lax
from jax.experimental import pallas as pl
from jax.experimental.pallas import tpu as pltpu
from jax.experimental.pallas import tpu_sc as plsc

E = 320000
D = 128
PD = 64
N = 10000

NSEG = 2
ESEG = E // NSEG
NC = 2
NS = 16
NW = NC * NS
EPW = ESEG // NW
CHUNK = 40
NCHUNK = EPW // CHUNK
NBUF = 5
RPS = 624
TAIL = N - NS * RPS

GATE_B = 16000


def _gate_body(lp_ref, x_ref, wt_ref, out_ref):
    x = x_ref[...]
    a = x[:, :PD]
    b = x[:, PD:]
    wt = wt_ref[...]
    ga = jnp.clip(lax.dot(a, wt, preferred_element_type=jnp.float32), 0.0, 1.0)
    gb = jnp.clip(lax.dot(b, wt, preferred_element_type=jnp.float32), 0.0, 1.0)
    at = jnp.concatenate([a, a], axis=1)
    bt = jnp.concatenate([b, b], axis=1)
    lp = jnp.abs(lp_ref[0])
    out_ref[...] = lp * (at * ga + bt * gb) + x


def _gate(x, wt, lp, seg):
    nblk = ESEG // GATE_B
    return pl.pallas_call(
        _gate_body,
        grid=(nblk,),
        in_specs=[
            pl.BlockSpec(memory_space=pltpu.SMEM),
            pl.BlockSpec((GATE_B, D), lambda i, _o=seg * nblk: (i + _o, 0)),
            pl.BlockSpec((PD, D), lambda i: (0, 0)),
        ],
        out_specs=pl.BlockSpec((GATE_B, D), lambda i: (i, 0)),
        out_shape=jax.ShapeDtypeStruct((ESEG, D), jnp.float32),
    )(lp, x, wt)


def _sc_scatter(comb, idx, zeros, seg):
    mesh = plsc.VectorSubcoreMesh(core_axis_name="c", subcore_axis_name="s")
    ibase0 = seg * ESEG

    @functools.partial(
        pl.kernel,
        mesh=mesh,
        out_type=jax.ShapeDtypeStruct((NC * N, D), jnp.float32),
        scratch_types=(
            [pltpu.VMEM((CHUNK, D), jnp.float32) for _ in range(NBUF)]
            + [pltpu.VMEM((CHUNK,), jnp.int32) for _ in range(NBUF)]
            + [pltpu.VMEM_SHARED((N, D), jnp.float32)]
            + [pltpu.SemaphoreType.DMA for _ in range(NBUF)]
        ),
    )
    def run(comb_hbm, idx_hbm, zeros_hbm, out_hbm,
            eb0, eb1, eb2, eb3, eb4, ib0, ib1, ib2, ib3, ib4,
            acc, sg0, sg1, sg2, sg3, sg4):
        ebufs = [eb0, eb1, eb2, eb3, eb4]
        ibufs = [ib0, ib1, ib2, ib3, ib4]
        sgs = [sg0, sg1, sg2, sg3, sg4]
        c = lax.axis_index("c")
        s = lax.axis_index("s")
        wid = s * NC + c
        base = wid * EPW
        ibase = ibase0 + base

        for b in range(NBUF):
            off = b * CHUNK
            pltpu.async_copy(comb_hbm.at[pl.ds(base + off, CHUNK)],
                             ebufs[b], sgs[b])
            pltpu.async_copy(idx_hbm.at[pl.ds(ibase + off, CHUNK)],
                             ibufs[b], sgs[b])

        pltpu.sync_copy(zeros_hbm.at[pl.ds(s * RPS, RPS)],
                        acc.at[pl.ds(s * RPS, RPS)])

        @pl.when(s == 0)
        def _():
            pltpu.sync_copy(zeros_hbm.at[pl.ds(NS * RPS, TAIL)],
                            acc.at[pl.ds(NS * RPS, TAIL)])

        plsc.subcore_barrier()

        def body(g, carry):
            for b in range(NBUF):
                i = g * NBUF + b
                pltpu.make_async_copy(
                    comb_hbm.at[pl.ds(base, CHUNK)], ebufs[b], sgs[b]).wait()
                pltpu.make_async_copy(
                    idx_hbm.at[pl.ds(ibase, CHUNK)], ibufs[b], sgs[b]).wait()
                pltpu.sync_copy(ebufs[b], acc.at[ibufs[b]], add=True)
                nxt = i + NBUF

                @pl.when(nxt < NCHUNK)
                def _():
                    off = nxt * CHUNK
                    pltpu.async_copy(comb_hbm.at[pl.ds(base + off, CHUNK)],
                                     ebufs[b], sgs[b])
                    pltpu.async_copy(idx_hbm.at[pl.ds(ibase + off, CHUNK)],
                                     ibufs[b], sgs[b])
            return carry

        lax.fori_loop(0, NCHUNK // NBUF, body, 0)
        plsc.subcore_barrier()
        pltpu.sync_copy(acc.at[pl.ds(s * RPS, RPS)],
                        out_hbm.at[pl.ds(c * N + s * RPS, RPS)])

        @pl.when(s == 0)
        def _():
            pltpu.sync_copy(acc.at[pl.ds(NS * RPS, TAIL)],
                            out_hbm.at[pl.ds(c * N + NS * RPS, TAIL)])

    return run(comb, idx, zeros)


def _add_body(*refs):
    o_ref = refs[-1]
    acc = refs[0][...]
    for r in refs[1:-1]:
        acc = acc + r[...]
    o_ref[...] = acc


def _final_add(partials_list):
    bn = 2000
    nblk = N // bn
    in_specs = []
    args = []
    for p in partials_list:
        in_specs.append(pl.BlockSpec((bn, D), lambda i: (i, 0)))
        in_specs.append(pl.BlockSpec((bn, D), lambda i, _o=nblk: (i + _o, 0)))
        args += [p, p]
    return pl.pallas_call(
        _add_body,
        grid=(nblk,),
        in_specs=in_specs,
        out_specs=pl.BlockSpec((bn, D), lambda i: (i, 0)),
        out_shape=jax.ShapeDtypeStruct((N, D), jnp.float32),
    )(*args)


def kernel(x, index, dim, dim_size, W, learnable_param):
    del dim, dim_size
    wt = W.T
    idx = index.astype(jnp.int32)
    zeros = jnp.zeros((N, D), jnp.float32)
    partials = []
    for k in range(NSEG):
        comb_k = _gate(x, wt, learnable_param, k)
        partials.append(_sc_scatter(comb_k, idx, zeros, k))
    return _final_add(partials)
